# baseline (speedup 1.0000x reference)
import functools

import jax
import jax.numpy as jnp
from jax import lax
from jax.experimental import pallas as pl
from jax.experimental.pallas import tpu as pltpu
from jax.experimental.pallas import tpu_sc as plsc

_SLOTS = 100
_R = 100000
_N = 16384
_D = 128
_SEG = 6272
_OWNER_SZ = 16 * _SEG
_NC = 2
_NS = 16
_NW = _NC * _NS
_RD_PER_W = _N // _NW
_CH = 128
_NCH = _RD_PER_W // _CH
_WR_VREGS = _N // 16


def _sc_body(mem_hbm, img_hbm, tgt_hbm, slot_hbm, ridx_hbm,
             out_hbm, rt_hbm,
             owner_sp, oseg_v, tgt_v, slot_v, ridx_v, own_v, ownc_v,
             mrows_v, arows_v, rt_v, tmp16_v, sem):
  cid = lax.axis_index("c")
  sid = lax.axis_index("s")
  wid = sid * _NC + cid
  lane = lax.iota(jnp.int32, 16)
  nxt_idx = jnp.minimum(lane + 1, 15)
  is_last = lane == 15

  lo = sid * _SEG

  def init_body(t, carry):
    oseg_v[pl.ds(t * 16, 16)] = jnp.full((16,), -1, jnp.int32)
    return carry
  lax.fori_loop(0, _SEG // 16, init_body, None)

  pltpu.sync_copy(tgt_hbm, tgt_v)
  pltpu.sync_copy(slot_hbm, slot_v)

  def scat_body(k, carry):
    t = tgt_v[pl.ds(k * 16, 16)]
    s = slot_v[pl.ds(k * 16, 16)]
    r = t * _SLOTS + s
    key = (r << 4) | lane
    ival = k * 16 + lane
    sk, sv = plsc.sort_key_val(key, ival)
    sr = sk >> 4
    tmp16_v[...] = sk
    nxt = plsc.load_gather(tmp16_v, [nxt_idx]) >> 4
    keep = (nxt != sr) | is_last
    m = keep & (sr >= lo) & (sr < lo + _SEG)
    addr = jnp.where(m, sr - lo, 0)
    plsc.store_scatter(oseg_v, [addr], sv, mask=m)
    return carry
  lax.fori_loop(0, _WR_VREGS, scat_body, None)

  pltpu.sync_copy(oseg_v, owner_sp.at[pl.ds(lo, _SEG)])
  plsc.subcore_barrier()

  rbase = wid * _RD_PER_W
  colidx = [(lane + (16 * c + 127)) % 128 for c in range(8)]

  for j in range(_NCH):
    b = rbase + j * _CH
    pltpu.sync_copy(ridx_hbm.at[pl.ds(b, _CH)], ridx_v.at[j])
    pltpu.async_copy(owner_sp.at[ridx_v.at[j]], own_v, sem).wait()
    pltpu.async_copy(mem_hbm.at[ridx_v.at[j]], mrows_v, sem).wait()
    for t in range(_CH // 16):
      o = own_v[pl.ds(t * 16, 16)]
      ownc_v[pl.ds(t * 16, 16)] = jnp.maximum(o, 0)
      rt_v[pl.ds(t * 16, 16)] = ridx_v[j, pl.ds(t * 16, 16)] // _SLOTS
    pltpu.async_copy(img_hbm.at[ownc_v], arows_v, sem).wait()

    def row_body(rr, carry):
      rsplat = jnp.zeros((16,), jnp.int32) + rr
      ob = plsc.load_gather(own_v, [rsplat])
      msk = ob >= 0
      for c in range(8):
        a = plsc.load_gather(arows_v, [rsplat, colidx[c]])
        mm = mrows_v[rr, pl.ds(16 * c, 16)]
        mrows_v[rr, pl.ds(16 * c, 16)] = jnp.where(msk, a, mm)
      return carry
    lax.fori_loop(0, _CH, row_body, None)

    pltpu.sync_copy(mrows_v, out_hbm.at[pl.ds(b, _CH)])
    pltpu.sync_copy(rt_v, rt_hbm.at[pl.ds(b, _CH)])


@jax.jit
def _sc_call(mem, images, targets, slot_idx, read_idx):
  mesh = plsc.VectorSubcoreMesh(core_axis_name="c", subcore_axis_name="s")
  f = functools.partial(
      pl.kernel, _sc_body, mesh=mesh,
      out_type=(jax.ShapeDtypeStruct((_N, _D), jnp.float32),
                jax.ShapeDtypeStruct((_N,), jnp.int32)),
      scratch_types=[
          pltpu.VMEM_SHARED((_OWNER_SZ,), jnp.int32),
          pltpu.VMEM((_SEG,), jnp.int32),
          pltpu.VMEM((_N,), jnp.int32),
          pltpu.VMEM((_N,), jnp.int32),
          pltpu.VMEM((_NCH, _CH), jnp.int32),
          pltpu.VMEM((_CH,), jnp.int32),
          pltpu.VMEM((_CH,), jnp.int32),
          pltpu.VMEM((_CH, _D), jnp.float32),
          pltpu.VMEM((_CH, _D), jnp.float32),
          pltpu.VMEM((_CH,), jnp.int32),
          pltpu.VMEM((16,), jnp.int32),
          pltpu.SemaphoreType.DMA,
      ],
      compiler_params=pltpu.CompilerParams(needs_layout_passes=False))
  return f()(mem, images, targets, slot_idx, read_idx)


def kernel(mem, images, targets, slot_idx, read_idx):
  out_vals, out_tgt = _sc_call(mem, images, targets, slot_idx, read_idx)
  return out_vals, out_tgt
